# baseline (speedup 1.0000x reference)
import functools

import jax
import jax.numpy as jnp
from jax import lax
from jax.experimental import pallas as pl
from jax.experimental.pallas import tpu as pltpu
from jax.experimental.pallas import tpu_sc as plsc

N = 10000
E = 320000
D_IN = 128
D_HID = 256
D_OUT = 128

NC = 2
NS = 16
NW = NC * NS
C = 80
NCH = E // C
CPW = NCH // NW
NR = 5
RCH = CPW // NR
RPT = 624
TAIL = N - NS * RPT
TAIL_OFF = NS * RPT

_mesh = plsc.VectorSubcoreMesh(
    core_axis_name="c", subcore_axis_name="s", num_cores=NC, num_subcores=NS)


HR = 80
HF = HR * 128
HCOL = HF // 5


@functools.partial(
    pl.kernel,
    out_type=jax.ShapeDtypeStruct((2 * NC, HR, 128), jnp.float32),
    mesh=_mesh,
    scratch_types=[
        pltpu.VMEM((RCH, C), jnp.int32),
        pltpu.VMEM((RCH, C), jnp.int32),
        pltpu.VMEM((HF,), jnp.float32),
        pltpu.VMEM((HF,), jnp.float32),
        pltpu.VMEM((HCOL,), jnp.float32),
        pltpu.VMEM((16, 128), jnp.float32),
        pltpu.VMEM_SHARED((NS, HF), jnp.float32),
        pltpu.VMEM_SHARED((NS, HF), jnp.float32),
    ],
    compiler_params=pltpu.CompilerParams(needs_layout_passes=False),
)
def _deg_kernel(src_hbm, dst_hbm, zeros1_hbm, zeros_hbm, out_hbm,
                src_v, dst_v, hsrc_v, hdst_v, tmp_v, acc_v,
                stgs_sh, stgd_sh):
    c = lax.axis_index("c")
    s = lax.axis_index("s")
    w = c * NS + s

    pltpu.sync_copy(zeros1_hbm, hsrc_v)
    pltpu.sync_copy(zeros1_hbm, hdst_v)

    ones16 = jnp.ones((16,), jnp.float32)
    for r in range(NR):
        pltpu.sync_copy(src_hbm.at[w * NR + r], src_v)
        pltpu.sync_copy(dst_hbm.at[w * NR + r], dst_v)

        def body(j, carry):
            for k in range(C // 16):
                iv = src_v[j, pl.ds(k * 16, 16)]
                plsc.addupdate_scatter(hsrc_v, [iv], ones16)
                jv = dst_v[j, pl.ds(k * 16, 16)]
                plsc.addupdate_scatter(hdst_v, [jv], ones16)
            return carry

        lax.fori_loop(0, RCH, body, 0)

    pltpu.sync_copy(hsrc_v, stgs_sh.at[s])
    pltpu.sync_copy(hdst_v, stgd_sh.at[s])
    plsc.subcore_barrier()

    def reduce_hist(stg, out_k, ti):
        base = ti * HCOL
        pltpu.sync_copy(zeros_hbm.at[pl.ds(0, 16)], acc_v)

        def red(t, carry):
            pltpu.sync_copy(stg.at[t, pl.ds(base, HCOL)], tmp_v)
            for i in range(HCOL // 16):
                rr, cc = i // 8, (i % 8) * 16
                acc_v[rr, pl.ds(cc, 16)] = (acc_v[rr, pl.ds(cc, 16)]
                                            + tmp_v[pl.ds(i * 16, 16)])
            return carry

        lax.fori_loop(0, NS, red, 0)
        pltpu.sync_copy(acc_v, out_hbm.at[out_k, pl.ds(ti * 16, 16)])

    @pl.when(s < 5)
    def _():
        reduce_hist(stgs_sh, 2 * c, s)

    @pl.when(jnp.logical_and(s >= 8, s < 13))
    def _():
        reduce_hist(stgd_sh, 2 * c + 1, s - 8)


@functools.partial(
    pl.kernel,
    out_type=jax.ShapeDtypeStruct((NC, N, D_IN), jnp.float32),
    mesh=_mesh,
    scratch_types=[
        pltpu.VMEM((RCH, C), jnp.int32),
        pltpu.VMEM((RCH, C), jnp.int32),
        pltpu.VMEM((C, D_IN), jnp.float32),
        pltpu.VMEM((C, D_IN), jnp.float32),
        pltpu.VMEM_SHARED((N, D_IN), jnp.float32),
        pltpu.SemaphoreType.DMA,
        pltpu.SemaphoreType.DMA,
    ],
)
def _agg_kernel(z_hbm, src_hbm, dst_hbm, zeros_hbm, out_hbm,
                src_v, dst_v, rows_a, rows_b, accum_sh, sem_a, sem_b):
    c = lax.axis_index("c")
    s = lax.axis_index("s")
    w = c * NS + s
    pltpu.sync_copy(zeros_hbm.at[pl.ds(0, RPT)], accum_sh.at[pl.ds(s * RPT, RPT)])

    @pl.when(s == NS - 1)
    def _():
        pltpu.sync_copy(zeros_hbm.at[pl.ds(0, TAIL)],
                        accum_sh.at[pl.ds(TAIL_OFF, TAIL)])

    plsc.subcore_barrier()

    for r in range(NR):
        pltpu.sync_copy(src_hbm.at[w * NR + r], src_v)
        pltpu.sync_copy(dst_hbm.at[w * NR + r], dst_v)
        pltpu.async_copy(z_hbm.at[src_v.at[0]], rows_a, sem_a)

        def body(g, carry):
            j0 = 2 * g
            pltpu.async_copy(z_hbm.at[src_v.at[j0 + 1]], rows_b, sem_b)
            pltpu.make_async_copy(z_hbm.at[src_v.at[j0]], rows_a, sem_a).wait()
            pltpu.sync_copy(rows_a, accum_sh.at[dst_v.at[j0]], add=True)
            pltpu.async_copy(z_hbm.at[src_v.at[j0 + 2]], rows_a, sem_a)
            pltpu.make_async_copy(z_hbm.at[src_v.at[j0 + 1]], rows_b, sem_b).wait()
            pltpu.sync_copy(rows_b, accum_sh.at[dst_v.at[j0 + 1]], add=True)
            return carry

        lax.fori_loop(0, (RCH - 1) // 2, body, 0)
        pltpu.make_async_copy(z_hbm.at[src_v.at[RCH - 1]], rows_a, sem_a).wait()
        pltpu.sync_copy(rows_a, accum_sh.at[dst_v.at[RCH - 1]], add=True)
    plsc.subcore_barrier()
    pltpu.sync_copy(accum_sh.at[pl.ds(s * RPT, RPT)],
                    out_hbm.at[c, pl.ds(s * RPT, RPT)])

    @pl.when(s == NS - 1)
    def _():
        pltpu.sync_copy(accum_sh.at[pl.ds(TAIL_OFF, TAIL)],
                        out_hbm.at[c, pl.ds(TAIL_OFF, TAIL)])


def _norms(degs_ref):
    d = degs_ref[...]
    deg_out = d[0] + d[2]
    deg_in = d[1] + d[3]
    norm_src = lax.rsqrt(jnp.maximum(deg_out, 1.0))
    norm_dst = lax.rsqrt(jnp.maximum(deg_in, 1.0))
    return norm_src, norm_dst


def _scale_body(x_ref, degs_ref, o_ref):
    norm_src, _ = _norms(degs_ref)
    o_ref[...] = x_ref[...] * norm_src[:, None]


def _mid_body(aggp_ref, degs_ref, w1_ref, b1_ref, w2_ref, o_ref):
    norm_src, norm_dst = _norms(degs_ref)
    agg = (aggp_ref[0] + aggp_ref[1]) * norm_dst[:, None]
    h1 = jnp.maximum(
        jnp.dot(agg, w1_ref[...], preferred_element_type=jnp.float32)
        + b1_ref[...], 0.0)
    o_ref[...] = jnp.dot(h1 * norm_src[:, None], w2_ref[...],
                         preferred_element_type=jnp.float32)


def _fin_body(aggp_ref, degs_ref, b2_ref, o_ref):
    _, norm_dst = _norms(degs_ref)
    agg = (aggp_ref[0] + aggp_ref[1]) * norm_dst[:, None]
    o_ref[...] = jnp.maximum(agg + b2_ref[...], 0.0)


_RB = 2048
_GRID = 5


def _tc_scale(x, degs):
    return pl.pallas_call(
        _scale_body,
        grid=(_GRID,),
        in_specs=[
            pl.BlockSpec((_RB, D_IN), lambda i: (i, 0)),
            pl.BlockSpec((4, _RB), lambda i: (0, i)),
        ],
        out_specs=pl.BlockSpec((_RB, D_IN), lambda i: (i, 0)),
        out_shape=jax.ShapeDtypeStruct((N, D_IN), jnp.float32),
    )(x, degs)


def _tc_mid(aggp, degs, W1, b1, W2):
    return pl.pallas_call(
        _mid_body,
        grid=(_GRID,),
        in_specs=[
            pl.BlockSpec((NC, _RB, D_IN), lambda i: (0, i, 0)),
            pl.BlockSpec((4, _RB), lambda i: (0, i)),
            pl.BlockSpec((D_IN, D_HID), lambda i: (0, 0)),
            pl.BlockSpec((1, D_HID), lambda i: (0, 0)),
            pl.BlockSpec((D_HID, D_OUT), lambda i: (0, 0)),
        ],
        out_specs=pl.BlockSpec((_RB, D_OUT), lambda i: (i, 0)),
        out_shape=jax.ShapeDtypeStruct((N, D_OUT), jnp.float32),
    )(aggp, degs, W1, b1, W2)


def _tc_fin(aggp, degs, b2):
    return pl.pallas_call(
        _fin_body,
        grid=(_GRID,),
        in_specs=[
            pl.BlockSpec((NC, _RB, D_OUT), lambda i: (0, i, 0)),
            pl.BlockSpec((4, _RB), lambda i: (0, i)),
            pl.BlockSpec((1, D_OUT), lambda i: (0, 0)),
        ],
        out_specs=pl.BlockSpec((_RB, D_OUT), lambda i: (i, 0)),
        out_shape=jax.ShapeDtypeStruct((N, D_OUT), jnp.float32),
    )(aggp, degs, b2)


def kernel(x, edge_index, W1, b1, W2, b2):
    srcf = edge_index[0].astype(jnp.int32)
    dstf = edge_index[1].astype(jnp.int32)
    src3 = srcf.reshape(NW * NR, RCH, C)
    dst3 = dstf.reshape(NW * NR, RCH, C)
    zeros_flat = jnp.zeros((HF,), jnp.float32)
    zeros_rows = jnp.zeros((RPT, D_IN), jnp.float32)
    b1r = b1.reshape(1, D_HID)
    b2r = b2.reshape(1, D_OUT)

    degs4 = _deg_kernel(src3, dst3, zeros_flat, zeros_rows)
    degs = degs4.reshape(2 * NC, HF)

    h0 = _tc_scale(x, degs)
    agg1p = _agg_kernel(h0, src3, dst3, zeros_rows)
    z2 = _tc_mid(agg1p, degs, W1, b1r, W2)
    agg2p = _agg_kernel(z2, src3, dst3, zeros_rows)
    return _tc_fin(agg2p, degs, b2r)


# speedup vs baseline: 1.1219x; 1.1219x over previous
import functools

import jax
import jax.numpy as jnp
from jax import lax
from jax.experimental import pallas as pl
from jax.experimental.pallas import tpu as pltpu
from jax.experimental.pallas import tpu_sc as plsc

N = 10000
E = 320000
D_IN = 128
D_HID = 256
D_OUT = 128

NC = 2
NS = 16
NW = NC * NS
C = 80
NCH = E // C
CPW = NCH // NW
NR = 5
RCH = CPW // NR
RPT = 624
TAIL = N - NS * RPT
TAIL_OFF = NS * RPT

_mesh = plsc.VectorSubcoreMesh(
    core_axis_name="c", subcore_axis_name="s", num_cores=NC, num_subcores=NS)


HR = 80
HF = HR * 128
HCOL = HF // 5


@functools.partial(
    pl.kernel,
    out_type=jax.ShapeDtypeStruct((2 * NC, HR, 128), jnp.float32),
    mesh=_mesh,
    scratch_types=[
        pltpu.VMEM((RCH, C), jnp.int32),
        pltpu.VMEM((RCH, C), jnp.int32),
        pltpu.VMEM((HF,), jnp.float32),
        pltpu.VMEM((HF,), jnp.float32),
        pltpu.VMEM((HCOL,), jnp.float32),
        pltpu.VMEM((16, 128), jnp.float32),
        pltpu.VMEM_SHARED((NS, HF), jnp.float32),
        pltpu.VMEM_SHARED((NS, HF), jnp.float32),
    ],
    compiler_params=pltpu.CompilerParams(needs_layout_passes=False),
)
def _deg_kernel(src_hbm, dst_hbm, zeros1_hbm, zeros_hbm, out_hbm,
                src_v, dst_v, hsrc_v, hdst_v, tmp_v, acc_v,
                stgs_sh, stgd_sh):
    c = lax.axis_index("c")
    s = lax.axis_index("s")
    w = c * NS + s

    pltpu.sync_copy(zeros1_hbm, hsrc_v)
    pltpu.sync_copy(zeros1_hbm, hdst_v)

    ones16 = jnp.ones((16,), jnp.float32)
    for r in range(NR):
        pltpu.sync_copy(src_hbm.at[w * NR + r], src_v)
        pltpu.sync_copy(dst_hbm.at[w * NR + r], dst_v)

        def body(j, carry):
            for k in range(C // 16):
                iv = src_v[j, pl.ds(k * 16, 16)]
                plsc.addupdate_scatter(hsrc_v, [iv], ones16)
                jv = dst_v[j, pl.ds(k * 16, 16)]
                plsc.addupdate_scatter(hdst_v, [jv], ones16)
            return carry

        lax.fori_loop(0, RCH, body, 0)

    pltpu.sync_copy(hsrc_v, stgs_sh.at[s])
    pltpu.sync_copy(hdst_v, stgd_sh.at[s])
    plsc.subcore_barrier()

    def reduce_hist(stg, out_k, ti):
        base = ti * HCOL
        pltpu.sync_copy(zeros_hbm.at[pl.ds(0, 16)], acc_v)

        def red(t, carry):
            pltpu.sync_copy(stg.at[t, pl.ds(base, HCOL)], tmp_v)
            for i in range(HCOL // 16):
                rr, cc = i // 8, (i % 8) * 16
                acc_v[rr, pl.ds(cc, 16)] = (acc_v[rr, pl.ds(cc, 16)]
                                            + tmp_v[pl.ds(i * 16, 16)])
            return carry

        lax.fori_loop(0, NS, red, 0)
        pltpu.sync_copy(acc_v, out_hbm.at[out_k, pl.ds(ti * 16, 16)])

    @pl.when(s < 5)
    def _():
        reduce_hist(stgs_sh, 2 * c, s)

    @pl.when(jnp.logical_and(s >= 8, s < 13))
    def _():
        reduce_hist(stgd_sh, 2 * c + 1, s - 8)


@functools.partial(
    pl.kernel,
    out_type=jax.ShapeDtypeStruct((NC, N, D_IN), jnp.float32),
    mesh=_mesh,
    scratch_types=[
        pltpu.VMEM((RCH, C), jnp.int32),
        pltpu.VMEM((RCH, C), jnp.int32),
        pltpu.VMEM((C, D_IN), jnp.float32),
        pltpu.VMEM((C, D_IN), jnp.float32),
        pltpu.VMEM((C, D_IN), jnp.float32),
        pltpu.VMEM_SHARED((N, D_IN), jnp.float32),
        pltpu.SemaphoreType.DMA,
        pltpu.SemaphoreType.DMA,
        pltpu.SemaphoreType.DMA,
    ],
)
def _agg_kernel(z_hbm, src_hbm, dst_hbm, zeros_hbm, out_hbm,
                src_v, dst_v, rows_a, rows_b, rows_c, accum_sh,
                sem_a, sem_b, sem_c):
    c = lax.axis_index("c")
    s = lax.axis_index("s")
    w = c * NS + s
    pltpu.sync_copy(zeros_hbm.at[pl.ds(0, RPT)], accum_sh.at[pl.ds(s * RPT, RPT)])

    @pl.when(s == NS - 1)
    def _():
        pltpu.sync_copy(zeros_hbm.at[pl.ds(0, TAIL)],
                        accum_sh.at[pl.ds(TAIL_OFF, TAIL)])

    plsc.subcore_barrier()

    bufs = ((rows_a, sem_a), (rows_b, sem_b), (rows_c, sem_c))

    def gath(j, b):
        pltpu.async_copy(z_hbm.at[src_v.at[j]], bufs[b][0], bufs[b][1])

    def scat(j, b):
        pltpu.make_async_copy(z_hbm.at[src_v.at[j]],
                              bufs[b][0], bufs[b][1]).wait()
        pltpu.sync_copy(bufs[b][0], accum_sh.at[dst_v.at[j]], add=True)

    for r in range(NR):
        pltpu.sync_copy(src_hbm.at[w * NR + r], src_v)
        pltpu.sync_copy(dst_hbm.at[w * NR + r], dst_v)
        gath(0, 0)
        gath(1, 1)

        def body(g, carry):
            j0 = 3 * g
            gath(j0 + 2, 2)
            scat(j0, 0)
            gath(j0 + 3, 0)
            scat(j0 + 1, 1)
            gath(j0 + 4, 1)
            scat(j0 + 2, 2)
            return carry

        lax.fori_loop(0, 7, body, 0)
        gath(23, 2)
        scat(21, 0)
        gath(24, 0)
        scat(22, 1)
        scat(23, 2)
        scat(24, 0)
    plsc.subcore_barrier()
    pltpu.sync_copy(accum_sh.at[pl.ds(s * RPT, RPT)],
                    out_hbm.at[c, pl.ds(s * RPT, RPT)])

    @pl.when(s == NS - 1)
    def _():
        pltpu.sync_copy(accum_sh.at[pl.ds(TAIL_OFF, TAIL)],
                        out_hbm.at[c, pl.ds(TAIL_OFF, TAIL)])


def _norms(degs_ref):
    d = degs_ref[...]
    deg_out = d[0] + d[2]
    deg_in = d[1] + d[3]
    norm_src = lax.rsqrt(jnp.maximum(deg_out, 1.0))
    norm_dst = lax.rsqrt(jnp.maximum(deg_in, 1.0))
    return norm_src, norm_dst


def _scale_body(x_ref, degs_ref, o_ref):
    norm_src, _ = _norms(degs_ref)
    o_ref[...] = x_ref[...] * norm_src[:, None]


def _mid_body(aggp_ref, degs_ref, w1_ref, b1_ref, w2_ref, o_ref):
    norm_src, norm_dst = _norms(degs_ref)
    agg = (aggp_ref[0] + aggp_ref[1]) * norm_dst[:, None]
    h1 = jnp.maximum(
        jnp.dot(agg, w1_ref[...], preferred_element_type=jnp.float32)
        + b1_ref[...], 0.0)
    o_ref[...] = jnp.dot(h1 * norm_src[:, None], w2_ref[...],
                         preferred_element_type=jnp.float32)


def _fin_body(aggp_ref, degs_ref, b2_ref, o_ref):
    _, norm_dst = _norms(degs_ref)
    agg = (aggp_ref[0] + aggp_ref[1]) * norm_dst[:, None]
    o_ref[...] = jnp.maximum(agg + b2_ref[...], 0.0)


_RB = 2048
_GRID = 5


def _tc_scale(x, degs):
    return pl.pallas_call(
        _scale_body,
        grid=(_GRID,),
        in_specs=[
            pl.BlockSpec((_RB, D_IN), lambda i: (i, 0)),
            pl.BlockSpec((4, _RB), lambda i: (0, i)),
        ],
        out_specs=pl.BlockSpec((_RB, D_IN), lambda i: (i, 0)),
        out_shape=jax.ShapeDtypeStruct((N, D_IN), jnp.float32),
    )(x, degs)


def _tc_mid(aggp, degs, W1, b1, W2):
    return pl.pallas_call(
        _mid_body,
        grid=(_GRID,),
        in_specs=[
            pl.BlockSpec((NC, _RB, D_IN), lambda i: (0, i, 0)),
            pl.BlockSpec((4, _RB), lambda i: (0, i)),
            pl.BlockSpec((D_IN, D_HID), lambda i: (0, 0)),
            pl.BlockSpec((1, D_HID), lambda i: (0, 0)),
            pl.BlockSpec((D_HID, D_OUT), lambda i: (0, 0)),
        ],
        out_specs=pl.BlockSpec((_RB, D_OUT), lambda i: (i, 0)),
        out_shape=jax.ShapeDtypeStruct((N, D_OUT), jnp.float32),
    )(aggp, degs, W1, b1, W2)


def _tc_fin(aggp, degs, b2):
    return pl.pallas_call(
        _fin_body,
        grid=(_GRID,),
        in_specs=[
            pl.BlockSpec((NC, _RB, D_OUT), lambda i: (0, i, 0)),
            pl.BlockSpec((4, _RB), lambda i: (0, i)),
            pl.BlockSpec((1, D_OUT), lambda i: (0, 0)),
        ],
        out_specs=pl.BlockSpec((_RB, D_OUT), lambda i: (i, 0)),
        out_shape=jax.ShapeDtypeStruct((N, D_OUT), jnp.float32),
    )(aggp, degs, b2)


def kernel(x, edge_index, W1, b1, W2, b2):
    srcf = edge_index[0].astype(jnp.int32)
    dstf = edge_index[1].astype(jnp.int32)
    src3 = srcf.reshape(NW * NR, RCH, C)
    dst3 = dstf.reshape(NW * NR, RCH, C)
    zeros_flat = jnp.zeros((HF,), jnp.float32)
    zeros_rows = jnp.zeros((RPT, D_IN), jnp.float32)
    b1r = b1.reshape(1, D_HID)
    b2r = b2.reshape(1, D_OUT)

    degs4 = _deg_kernel(src3, dst3, zeros_flat, zeros_rows)
    degs = degs4.reshape(2 * NC, HF)

    h0 = _tc_scale(x, degs)
    agg1p = _agg_kernel(h0, src3, dst3, zeros_rows)
    z2 = _tc_mid(agg1p, degs, W1, b1r, W2)
    agg2p = _agg_kernel(z2, src3, dst3, zeros_rows)
    return _tc_fin(agg2p, degs, b2r)


# speedup vs baseline: 1.1293x; 1.0066x over previous
import functools

import jax
import jax.numpy as jnp
from jax import lax
from jax.experimental import pallas as pl
from jax.experimental.pallas import tpu as pltpu
from jax.experimental.pallas import tpu_sc as plsc

N = 10000
E = 320000
D_IN = 128
D_HID = 256
D_OUT = 128

NC = 2
NS = 16
NW = NC * NS
C = 80
NCH = E // C
CPW = NCH // NW
NR = 5
RCH = CPW // NR
RPT = 624
TAIL = N - NS * RPT
TAIL_OFF = NS * RPT

_mesh = plsc.VectorSubcoreMesh(
    core_axis_name="c", subcore_axis_name="s", num_cores=NC, num_subcores=NS)


HR = 80
HF = HR * 128
HCOL = HF // 5


@functools.partial(
    pl.kernel,
    out_type=jax.ShapeDtypeStruct((2 * NC, HR, 128), jnp.float32),
    mesh=_mesh,
    scratch_types=[
        pltpu.VMEM((RCH, C), jnp.int32),
        pltpu.VMEM((RCH, C), jnp.int32),
        pltpu.VMEM((HF,), jnp.float32),
        pltpu.VMEM((HF,), jnp.float32),
        pltpu.VMEM((HCOL,), jnp.float32),
        pltpu.VMEM((16, 128), jnp.float32),
        pltpu.VMEM_SHARED((NS, HF), jnp.float32),
        pltpu.VMEM_SHARED((NS, HF), jnp.float32),
    ],
    compiler_params=pltpu.CompilerParams(needs_layout_passes=False),
)
def _deg_kernel(src_hbm, dst_hbm, zeros1_hbm, zeros_hbm, out_hbm,
                src_v, dst_v, hsrc_v, hdst_v, tmp_v, acc_v,
                stgs_sh, stgd_sh):
    c = lax.axis_index("c")
    s = lax.axis_index("s")
    w = c * NS + s

    pltpu.sync_copy(zeros1_hbm, hsrc_v)
    pltpu.sync_copy(zeros1_hbm, hdst_v)

    ones16 = jnp.ones((16,), jnp.float32)
    for r in range(NR):
        pltpu.sync_copy(src_hbm.at[w * NR + r], src_v)
        pltpu.sync_copy(dst_hbm.at[w * NR + r], dst_v)

        def body(j, carry):
            for k in range(C // 16):
                iv = src_v[j, pl.ds(k * 16, 16)]
                plsc.addupdate_scatter(hsrc_v, [iv], ones16)
                jv = dst_v[j, pl.ds(k * 16, 16)]
                plsc.addupdate_scatter(hdst_v, [jv], ones16)
            return carry

        lax.fori_loop(0, RCH, body, 0)

    pltpu.sync_copy(hsrc_v, stgs_sh.at[s])
    pltpu.sync_copy(hdst_v, stgd_sh.at[s])
    plsc.subcore_barrier()

    def reduce_hist(stg, out_k, ti):
        base = ti * HCOL
        pltpu.sync_copy(zeros_hbm.at[pl.ds(0, 16)], acc_v)

        def red(t, carry):
            pltpu.sync_copy(stg.at[t, pl.ds(base, HCOL)], tmp_v)
            for i in range(HCOL // 16):
                rr, cc = i // 8, (i % 8) * 16
                acc_v[rr, pl.ds(cc, 16)] = (acc_v[rr, pl.ds(cc, 16)]
                                            + tmp_v[pl.ds(i * 16, 16)])
            return carry

        lax.fori_loop(0, NS, red, 0)
        pltpu.sync_copy(acc_v, out_hbm.at[out_k, pl.ds(ti * 16, 16)])

    @pl.when(s < 5)
    def _():
        reduce_hist(stgs_sh, 2 * c, s)

    @pl.when(jnp.logical_and(s >= 8, s < 13))
    def _():
        reduce_hist(stgd_sh, 2 * c + 1, s - 8)


@functools.partial(
    pl.kernel,
    out_type=jax.ShapeDtypeStruct((NC, N, D_IN), jnp.float32),
    mesh=_mesh,
    scratch_types=[
        pltpu.VMEM((RCH, C), jnp.int32),
        pltpu.VMEM((RCH, C), jnp.int32),
        pltpu.VMEM((C, D_IN), jnp.float32),
        pltpu.VMEM((C, D_IN), jnp.float32),
        pltpu.VMEM((C, D_IN), jnp.float32),
        pltpu.VMEM((C, D_IN), jnp.float32),
        pltpu.VMEM_SHARED((N, D_IN), jnp.float32),
        pltpu.SemaphoreType.DMA,
        pltpu.SemaphoreType.DMA,
        pltpu.SemaphoreType.DMA,
        pltpu.SemaphoreType.DMA,
    ],
)
def _agg_kernel(z_hbm, src_hbm, dst_hbm, zeros_hbm, out_hbm,
                src_v, dst_v, rows_a, rows_b, rows_c, rows_d, accum_sh,
                sem_a, sem_b, sem_c, sem_d):
    c = lax.axis_index("c")
    s = lax.axis_index("s")
    w = c * NS + s
    pltpu.sync_copy(zeros_hbm.at[pl.ds(0, RPT)], accum_sh.at[pl.ds(s * RPT, RPT)])

    @pl.when(s == NS - 1)
    def _():
        pltpu.sync_copy(zeros_hbm.at[pl.ds(0, TAIL)],
                        accum_sh.at[pl.ds(TAIL_OFF, TAIL)])

    plsc.subcore_barrier()

    bufs = ((rows_a, sem_a), (rows_b, sem_b), (rows_c, sem_c), (rows_d, sem_d))

    def gath(j, b):
        pltpu.async_copy(z_hbm.at[src_v.at[j]], bufs[b][0], bufs[b][1])

    def scat(j, b):
        pltpu.make_async_copy(z_hbm.at[src_v.at[j]],
                              bufs[b][0], bufs[b][1]).wait()
        pltpu.sync_copy(bufs[b][0], accum_sh.at[dst_v.at[j]], add=True)

    for r in range(NR):
        pltpu.sync_copy(src_hbm.at[w * NR + r], src_v)
        pltpu.sync_copy(dst_hbm.at[w * NR + r], dst_v)
        gath(0, 0)
        gath(1, 1)
        gath(2, 2)

        def body(g, carry):
            j0 = 4 * g
            gath(j0 + 3, 3)
            scat(j0, 0)
            gath(j0 + 4, 0)
            scat(j0 + 1, 1)
            gath(j0 + 5, 1)
            scat(j0 + 2, 2)
            gath(j0 + 6, 2)
            scat(j0 + 3, 3)
            return carry

        lax.fori_loop(0, 5, body, 0)
        gath(23, 3)
        scat(20, 0)
        gath(24, 0)
        scat(21, 1)
        scat(22, 2)
        scat(23, 3)
        scat(24, 0)
    plsc.subcore_barrier()
    pltpu.sync_copy(accum_sh.at[pl.ds(s * RPT, RPT)],
                    out_hbm.at[c, pl.ds(s * RPT, RPT)])

    @pl.when(s == NS - 1)
    def _():
        pltpu.sync_copy(accum_sh.at[pl.ds(TAIL_OFF, TAIL)],
                        out_hbm.at[c, pl.ds(TAIL_OFF, TAIL)])


def _norms(degs_ref):
    d = degs_ref[...]
    deg_out = d[0] + d[2]
    deg_in = d[1] + d[3]
    norm_src = lax.rsqrt(jnp.maximum(deg_out, 1.0))
    norm_dst = lax.rsqrt(jnp.maximum(deg_in, 1.0))
    return norm_src, norm_dst


def _scale_body(x_ref, degs_ref, o_ref):
    norm_src, _ = _norms(degs_ref)
    o_ref[...] = x_ref[...] * norm_src[:, None]


def _mid_body(aggp_ref, degs_ref, w1_ref, b1_ref, w2_ref, o_ref):
    norm_src, norm_dst = _norms(degs_ref)
    agg = (aggp_ref[0] + aggp_ref[1]) * norm_dst[:, None]
    h1 = jnp.maximum(
        jnp.dot(agg, w1_ref[...], preferred_element_type=jnp.float32)
        + b1_ref[...], 0.0)
    o_ref[...] = jnp.dot(h1 * norm_src[:, None], w2_ref[...],
                         preferred_element_type=jnp.float32)


def _fin_body(aggp_ref, degs_ref, b2_ref, o_ref):
    _, norm_dst = _norms(degs_ref)
    agg = (aggp_ref[0] + aggp_ref[1]) * norm_dst[:, None]
    o_ref[...] = jnp.maximum(agg + b2_ref[...], 0.0)


_RB = 2048
_GRID = 5


def _tc_scale(x, degs):
    return pl.pallas_call(
        _scale_body,
        grid=(_GRID,),
        in_specs=[
            pl.BlockSpec((_RB, D_IN), lambda i: (i, 0)),
            pl.BlockSpec((4, _RB), lambda i: (0, i)),
        ],
        out_specs=pl.BlockSpec((_RB, D_IN), lambda i: (i, 0)),
        out_shape=jax.ShapeDtypeStruct((N, D_IN), jnp.float32),
    )(x, degs)


def _tc_mid(aggp, degs, W1, b1, W2):
    return pl.pallas_call(
        _mid_body,
        grid=(_GRID,),
        in_specs=[
            pl.BlockSpec((NC, _RB, D_IN), lambda i: (0, i, 0)),
            pl.BlockSpec((4, _RB), lambda i: (0, i)),
            pl.BlockSpec((D_IN, D_HID), lambda i: (0, 0)),
            pl.BlockSpec((1, D_HID), lambda i: (0, 0)),
            pl.BlockSpec((D_HID, D_OUT), lambda i: (0, 0)),
        ],
        out_specs=pl.BlockSpec((_RB, D_OUT), lambda i: (i, 0)),
        out_shape=jax.ShapeDtypeStruct((N, D_OUT), jnp.float32),
    )(aggp, degs, W1, b1, W2)


def _tc_fin(aggp, degs, b2):
    return pl.pallas_call(
        _fin_body,
        grid=(_GRID,),
        in_specs=[
            pl.BlockSpec((NC, _RB, D_OUT), lambda i: (0, i, 0)),
            pl.BlockSpec((4, _RB), lambda i: (0, i)),
            pl.BlockSpec((1, D_OUT), lambda i: (0, 0)),
        ],
        out_specs=pl.BlockSpec((_RB, D_OUT), lambda i: (i, 0)),
        out_shape=jax.ShapeDtypeStruct((N, D_OUT), jnp.float32),
    )(aggp, degs, b2)


def kernel(x, edge_index, W1, b1, W2, b2):
    srcf = edge_index[0].astype(jnp.int32)
    dstf = edge_index[1].astype(jnp.int32)
    src3 = srcf.reshape(NW * NR, RCH, C)
    dst3 = dstf.reshape(NW * NR, RCH, C)
    zeros_flat = jnp.zeros((HF,), jnp.float32)
    zeros_rows = jnp.zeros((RPT, D_IN), jnp.float32)
    b1r = b1.reshape(1, D_HID)
    b2r = b2.reshape(1, D_OUT)

    degs4 = _deg_kernel(src3, dst3, zeros_flat, zeros_rows)
    degs = degs4.reshape(2 * NC, HF)

    h0 = _tc_scale(x, degs)
    agg1p = _agg_kernel(h0, src3, dst3, zeros_rows)
    z2 = _tc_mid(agg1p, degs, W1, b1r, W2)
    agg2p = _agg_kernel(z2, src3, dst3, zeros_rows)
    return _tc_fin(agg2p, degs, b2r)
